# baseline (speedup 1.0000x reference)
import functools
import math

import jax
import jax.numpy as jnp
from jax import lax
from jax.experimental import pallas as pl
from jax.experimental.pallas import tpu as pltpu
from jax.experimental.pallas import tpu_sc as plsc

_B = 16384
_D = 128
_C = 100
_NB = 64
_CP = 112
_L = 16
_NC, _NS = 2, 16
_NW = _NC * _NS
_BPW = _B // _NW
_CH = 128
_NCH = _BPW // _CH
_GPC = _CH // _L
_P1 = _D - _L
_NACC = 4
_K = -_D * math.log(2.0 * math.pi)
_TCB = 2048


def _tc_body(zT_ref, out_ref):
    x = zT_ref[...]
    ones = jnp.full((_D, 1), -0.5, jnp.float32)
    out_ref[...] = jax.lax.dot_general(
        x * x, ones, (((1,), (0,)), ((), ())),
        preferred_element_type=jnp.float32)


_tc_call = pl.pallas_call(
    _tc_body,
    out_shape=jax.ShapeDtypeStruct((_B, 1), jnp.float32),
    grid=(_B // _TCB,),
    in_specs=[pl.BlockSpec((_TCB, _D), lambda i: (i, 0))],
    out_specs=pl.BlockSpec((_TCB, 1), lambda i: (i, 0)),
)


def _body(z0f_hbm, meansf_hbm, lvf_hbm, logcb_hbm, ct_hbm, bt_hbm,
          tp_hbm, out_hbm,
          mf_v, lv_v, pk_v, a_v, ct_v, bt_v, tp_v,
          z0b0, z0b1, out_v, sem0, sem1, semt):
    wid = lax.axis_index("s") * _NC + lax.axis_index("c")
    base = wid * _BPW

    bufs = ((z0b0, sem0), (z0b1, sem1))

    def start_chunk(c, slot):
        off = (base + c * _CH) * _D
        z0b, sem = bufs[slot]
        return pltpu.async_copy(z0f_hbm.at[pl.ds(off, _CH * _D)], z0b, sem)

    handles = [start_chunk(0, 0), None]

    th = [pltpu.async_copy(meansf_hbm, mf_v, semt),
          pltpu.async_copy(lvf_hbm, lv_v, semt),
          pltpu.async_copy(logcb_hbm, a_v, semt),
          pltpu.async_copy(ct_hbm.at[pl.ds(base, _BPW)], ct_v, semt),
          pltpu.async_copy(bt_hbm.at[pl.ds(base, _BPW)], bt_v, semt),
          pltpu.async_copy(tp_hbm.at[pl.ds(base, _BPW)], tp_v, semt)]
    for h in th:
        h.wait()

    lane = lax.iota(jnp.int32, _L)
    zero = jnp.zeros((_L,), jnp.float32)

    def pack_body(c, _):
        for j in range(_D // _L):
            sl = pl.ds(c * _D + j * _L, _L)
            pk_v[sl] = plsc.bitcast(
                plsc.pack(mf_v[sl], jnp.exp(-lv_v[sl]),
                          format=plsc.PackFormat.INTERLEAVED),
                jnp.int32)
        return 0

    lax.fori_loop(0, _C, pack_body, 0)

    nk = _CP // _L
    cbase = tuple((k * _L + lane) * _D for k in range(nk))

    def sum_body(d, carry):
        dl = carry[0]
        accs = [carry[1 + k] + plsc.load_gather(lv_v, [cbase[k] + dl])
                for k in range(nk)]
        return ((dl + 1) & (_D - 1), *accs)

    sums = lax.fori_loop(0, _D, sum_body, (lane,) + (zero,) * nk)
    for k in range(nk):
        sl = pl.ds(k * _L, _L)
        a_v[sl] = a_v[sl] - 0.5 * sums[1 + k]

    for c in range(_NCH):
        slot = c % 2
        if c + 1 < _NCH:
            handles[(c + 1) % 2] = start_chunk(c + 1, (c + 1) % 2)
        handles[slot].wait()
        z0b, _ = bufs[slot]

        def group_body(g, _, z0b=z0b, c=c):
            lo = c * _CH + g * _L
            ct = ct_v[pl.ds(lo, _L)]
            bt = bt_v[pl.ds(lo, _L)]
            rest = (plsc.load_gather(a_v, [ct])
                    + plsc.load_gather(a_v, [bt + _CP])
                    + (tp_v[pl.ds(lo, _L)] + jnp.float32(_K)))
            pv0 = (g * _L + lane) * _D + lane
            mi0 = ct * _D + lane
            qs = [zero] * _NACC

            def gstep(pv, mi, j):
                zg = plsc.load_gather(z0b, [pv])
                w = plsc.load_gather(pk_v, [mi])
                mu, iv = plsc.unpack(plsc.bitcast(w, jnp.bfloat16),
                                     format=plsc.PackFormat.INTERLEAVED)
                dlt = zg - mu
                a = j % _NACC
                qs[a] = qs[a] + dlt * dlt * iv

            for j in range(_P1):
                gstep(pv0 + j, mi0 + j, j)
            for j in range(_D - _P1):
                adj = jnp.where(lane >= _L - j, jnp.int32(_D), jnp.int32(0))
                gstep(pv0 + (_P1 + j) - adj, mi0 + (_P1 + j) - adj,
                      _P1 + j)

            q = (qs[0] + qs[1]) + (qs[2] + qs[3])
            out_v[pl.ds(lo, _L)] = -0.5 * q + rest
            return 0

        lax.fori_loop(0, _GPC, group_body, 0)

    pltpu.sync_copy(out_v, out_hbm.at[pl.ds(base, _BPW)])


_sc_call = pl.kernel(
    _body,
    out_type=jax.ShapeDtypeStruct((_B,), jnp.float32),
    mesh=plsc.VectorSubcoreMesh(core_axis_name="c", subcore_axis_name="s"),
    compiler_params=pltpu.CompilerParams(needs_layout_passes=False),
    scratch_types=[
        pltpu.VMEM((_C * _D,), jnp.float32),
        pltpu.VMEM((_C * _D,), jnp.float32),
        pltpu.VMEM((_C * _D,), jnp.int32),
        pltpu.VMEM((_CP + _NB,), jnp.float32),
        pltpu.VMEM((_BPW,), jnp.int32),
        pltpu.VMEM((_BPW,), jnp.int32),
        pltpu.VMEM((_BPW,), jnp.float32),
        pltpu.VMEM((_CH * _D,), jnp.float32),
        pltpu.VMEM((_CH * _D,), jnp.float32),
        pltpu.VMEM((_BPW,), jnp.float32),
        pltpu.SemaphoreType.DMA,
        pltpu.SemaphoreType.DMA,
        pltpu.SemaphoreType.DMA,
    ],
)


def kernel(z0, zT, means, log_vars, cell_probs, batch_probs,
           cell_type, batch_idx):
    tpart = _tc_call(zT).reshape(-1)
    z0f = z0.reshape(-1)
    meansf = means.reshape(-1)
    lvf = log_vars.reshape(-1)
    logcb = jnp.log(jnp.concatenate(
        [cell_probs, jnp.ones((_CP - _C,), jnp.float32), batch_probs]))
    ct = cell_type.astype(jnp.int32)
    bt = batch_idx.astype(jnp.int32)
    return _sc_call(z0f, meansf, lvf, logcb, ct, bt, tpart)


# speedup vs baseline: 1.2026x; 1.2026x over previous
import functools
import math

import jax
import jax.numpy as jnp
from jax import lax
from jax.experimental import pallas as pl
from jax.experimental.pallas import tpu as pltpu
from jax.experimental.pallas import tpu_sc as plsc

_B = 16384
_D = 128
_C = 100
_NB = 64
_CP = 112
_L = 16
_NC, _NS = 2, 16
_NW = _NC * _NS
_BPW = _B // _NW
_CH = 128
_NCH = _BPW // _CH
_GPC = _CH // _L
_P1 = _D - _L
_NACC = 2
_K = -_D * math.log(2.0 * math.pi)


def _body(z0f_hbm, zTf_hbm, meansf_hbm, lvf_hbm, logcb_hbm,
          ct_hbm, bt_hbm, out_hbm,
          mf_v, lv_v, pk_v, a_v, ct_v, bt_v,
          z0b0, z0b1, zTb0, zTb1, out_v, sem0, sem1, semt):
    wid = lax.axis_index("s") * _NC + lax.axis_index("c")
    base = wid * _BPW

    bufs = ((z0b0, zTb0, sem0), (z0b1, zTb1, sem1))

    def start_chunk(c, slot):
        off = (base + c * _CH) * _D
        z0b, zTb, sem = bufs[slot]
        h1 = pltpu.async_copy(z0f_hbm.at[pl.ds(off, _CH * _D)], z0b, sem)
        h2 = pltpu.async_copy(zTf_hbm.at[pl.ds(off, _CH * _D)], zTb, sem)
        return (h1, h2)

    handles = [start_chunk(0, 0), None]

    th = [pltpu.async_copy(meansf_hbm, mf_v, semt),
          pltpu.async_copy(lvf_hbm, lv_v, semt),
          pltpu.async_copy(logcb_hbm, a_v, semt),
          pltpu.async_copy(ct_hbm.at[pl.ds(base, _BPW)], ct_v, semt),
          pltpu.async_copy(bt_hbm.at[pl.ds(base, _BPW)], bt_v, semt)]
    for h in th:
        h.wait()

    lane = lax.iota(jnp.int32, _L)
    zero = jnp.zeros((_L,), jnp.float32)

    def pack_body(c, _):
        for j in range(_D // _L):
            sl = pl.ds(c * _D + j * _L, _L)
            pk_v[sl] = plsc.bitcast(
                plsc.pack(mf_v[sl], jnp.exp(-0.5 * lv_v[sl]),
                          format=plsc.PackFormat.INTERLEAVED),
                jnp.int32)
        return 0

    lax.fori_loop(0, _C, pack_body, 0)

    nk = _CP // _L
    cbase = tuple((k * _L + lane) * _D for k in range(nk))

    def sum_body(d, carry):
        dl = carry[0]
        accs = [carry[1 + k] + plsc.load_gather(lv_v, [cbase[k] + dl])
                for k in range(nk)]
        return ((dl + 1) & (_D - 1), *accs)

    sums = lax.fori_loop(0, _D, sum_body, (lane,) + (zero,) * nk)
    for k in range(nk):
        sl = pl.ds(k * _L, _L)
        a_v[sl] = a_v[sl] - 0.5 * sums[1 + k]

    for c in range(_NCH):
        slot = c % 2
        if c + 1 < _NCH:
            handles[(c + 1) % 2] = start_chunk(c + 1, (c + 1) % 2)
        h1, h2 = handles[slot]
        h1.wait()
        h2.wait()
        z0b, zTb, _ = bufs[slot]

        def group_body(g, _, z0b=z0b, zTb=zTb, c=c):
            lo = c * _CH + g * _L
            ct = ct_v[pl.ds(lo, _L)]
            bt = bt_v[pl.ds(lo, _L)]
            cat = (plsc.load_gather(a_v, [ct])
                   + plsc.load_gather(a_v, [bt + _CP]))
            pv0 = (g * _L + lane) * _D + lane
            mi0 = ct * _D + lane
            qs = [zero] * _NACC
            ts = [zero] * _NACC

            def gstep(pv, mi, j):
                zg = plsc.load_gather(z0b, [pv])
                tg = plsc.load_gather(zTb, [pv])
                w = plsc.load_gather(pk_v, [mi])
                mu, sd = plsc.unpack(plsc.bitcast(w, jnp.bfloat16),
                                     format=plsc.PackFormat.INTERLEAVED)
                u = (zg - mu) * sd
                a = j % _NACC
                qs[a] = qs[a] + u * u
                ts[a] = ts[a] + tg * tg

            for j in range(_P1):
                gstep(pv0 + j, mi0 + j, j)
            for j in range(_D - _P1):
                adj = jnp.where(lane >= _L - j, jnp.int32(_D), jnp.int32(0))
                gstep(pv0 + (_P1 + j) - adj, mi0 + (_P1 + j) - adj,
                      _P1 + j)

            q = qs[0] + qs[1]
            t = ts[0] + ts[1]
            out_v[pl.ds(lo, _L)] = (-0.5 * (q + t)
                                    + (cat + jnp.float32(_K)))
            return 0

        lax.fori_loop(0, _GPC, group_body, 0)

    pltpu.sync_copy(out_v, out_hbm.at[pl.ds(base, _BPW)])


_sc_call = pl.kernel(
    _body,
    out_type=jax.ShapeDtypeStruct((_B,), jnp.float32),
    mesh=plsc.VectorSubcoreMesh(core_axis_name="c", subcore_axis_name="s"),
    compiler_params=pltpu.CompilerParams(needs_layout_passes=False),
    scratch_types=[
        pltpu.VMEM((_C * _D,), jnp.float32),
        pltpu.VMEM((_C * _D,), jnp.float32),
        pltpu.VMEM((_C * _D,), jnp.int32),
        pltpu.VMEM((_CP + _NB,), jnp.float32),
        pltpu.VMEM((_BPW,), jnp.int32),
        pltpu.VMEM((_BPW,), jnp.int32),
        pltpu.VMEM((_CH * _D,), jnp.float32),
        pltpu.VMEM((_CH * _D,), jnp.float32),
        pltpu.VMEM((_CH * _D,), jnp.float32),
        pltpu.VMEM((_CH * _D,), jnp.float32),
        pltpu.VMEM((_BPW,), jnp.float32),
        pltpu.SemaphoreType.DMA,
        pltpu.SemaphoreType.DMA,
        pltpu.SemaphoreType.DMA,
    ],
)


def kernel(z0, zT, means, log_vars, cell_probs, batch_probs,
           cell_type, batch_idx):
    z0f = z0.reshape(-1)
    zTf = zT.reshape(-1)
    meansf = means.reshape(-1)
    lvf = log_vars.reshape(-1)
    logcb = jnp.log(jnp.concatenate(
        [cell_probs, jnp.ones((_CP - _C,), jnp.float32), batch_probs]))
    ct = cell_type.astype(jnp.int32)
    bt = batch_idx.astype(jnp.int32)
    return _sc_call(z0f, zTf, meansf, lvf, logcb, ct, bt)
